# baseline (speedup 1.0000x reference)
import functools

import jax
import jax.numpy as jnp
from jax import lax
from jax.experimental import pallas as pl
from jax.experimental.pallas import tpu as pltpu
from jax.experimental.pallas import tpu_sc as plsc

M = 512
KNN = 8
MRX = 10
MRT = 6

NW = 32
QTOT = 50000
QPW = 1568
NGROUP = QPW // 16

IDX_BITS = 9
IDX_MASK = (1 << IDX_BITS) - 1


def _freq_rows(nfreq):
    i = jax.lax.broadcasted_iota(jnp.int32, (1, nfreq), 1)
    return jnp.exp2(i.astype(jnp.float32))


def _prep_kernel(nodes_ref, t_ref, rrow_ref, W0_ref, b0_ref, W1_ref, b1_ref,
                 W2_ref, b2_ref, W3_ref, b3_ref, W4_ref, b4_ref, Wout_ref,
                 bout_ref, P_ref, invd_ref, nrep_ref):
    nodes = nodes_ref[...]
    nx = nodes[:, 0:1]
    ny = nodes[:, 1:2]
    nz = nodes[:, 2:3]
    fx = _freq_rows(MRX)
    ft = _freq_rows(MRT)
    t = t_ref[0, 0]
    t_col = jnp.full((M, 1), t, jnp.float32)

    parts = [nodes]
    for col in (nx, ny, nz):
        parts.append(jnp.sin(col * fx))
    for col in (nx, ny, nz):
        parts.append(jnp.cos(col * fx))
    parts.append(t_col)
    parts.append(jnp.sin(t_col * ft))
    parts.append(jnp.cos(t_col * ft))
    h0 = jnp.concatenate(parts, axis=1)

    dot = functools.partial(jnp.dot, preferred_element_type=jnp.float32,
                            precision=jax.lax.Precision.HIGHEST)
    h = jax.nn.relu(dot(h0, W0_ref[...]) + b0_ref[...])
    h = jax.nn.relu(dot(h, W1_ref[...]) + b1_ref[...])
    h = jax.nn.relu(dot(h, W2_ref[...]) + b2_ref[...])
    h = jnp.concatenate([h, h0], axis=1)
    h = jax.nn.relu(dot(h, W3_ref[...]) + b3_ref[...])
    h = jax.nn.relu(dot(h, W4_ref[...]) + b4_ref[...])
    out = dot(h, Wout_ref[...]) + bout_ref[...]

    dx = out[:, 0:1]
    dy = out[:, 1:2]
    dz = out[:, 2:3]
    r = out[:, 3:4] + 1.0
    i = out[:, 4:5]
    j = out[:, 5:6]
    k = out[:, 6:7]
    two_s = 2.0 / (r * r + i * i + j * j + k * k + 1e-8)
    R00 = 1.0 - two_s * (j * j + k * k)
    R01 = two_s * (i * j - k * r)
    R02 = two_s * (i * k + j * r)
    R10 = two_s * (i * j + k * r)
    R11 = 1.0 - two_s * (i * i + k * k)
    R12 = two_s * (j * k - i * r)
    R20 = two_s * (i * k - j * r)
    R21 = two_s * (j * k + i * r)
    R22 = 1.0 - two_s * (i * i + j * j)
    cx = nx + dx - (R00 * nx + R01 * ny + R02 * nz)
    cy = ny + dy - (R10 * nx + R11 * ny + R12 * nz)
    cz = nz + dz - (R20 * nx + R21 * ny + R22 * nz)
    P_ref[...] = jnp.concatenate(
        [R00, R01, R02, R10, R11, R12, R20, R21, R22, cx, cy, cz], axis=1)
    rrow = rrow_ref[...]
    invd_ref[...] = 1.0 / (2.0 * rrow * rrow + 1e-8)
    ones = jnp.ones((1, 16), jnp.float32)
    nrep_ref[...] = jnp.concatenate([nx * ones, ny * ones, nz * ones], axis=1)


def _prep(nodes, t, node_radius, W0, b0, W1, b1, W2, b2, W3, b3, W4, b4,
          Wout, bout):
    return pl.pallas_call(
        _prep_kernel,
        out_shape=(jax.ShapeDtypeStruct((M, 12), jnp.float32),
                   jax.ShapeDtypeStruct((1, M), jnp.float32),
                   jax.ShapeDtypeStruct((M, 48), jnp.float32)),
    )(nodes, t.reshape(1, 1), node_radius.reshape(1, -1), W0,
      b0.reshape(1, -1), W1, b1.reshape(1, -1), W2, b2.reshape(1, -1), W3,
      b3.reshape(1, -1), W4, b4.reshape(1, -1), Wout, bout.reshape(1, -1))


@functools.partial(
    pl.kernel,
    mesh=plsc.VectorSubcoreMesh(core_axis_name="c", subcore_axis_name="s"),
    compiler_params=pltpu.CompilerParams(needs_layout_passes=False),
    out_type=jax.ShapeDtypeStruct((QTOT * 3,), jnp.float32),
    scratch_types=[
        pltpu.VMEM((QPW * 3,), jnp.float32),
        pltpu.VMEM((QPW * 3,), jnp.float32),
        pltpu.VMEM((M * 3 * 16,), jnp.float32),
        pltpu.VMEM((M * 12,), jnp.float32),
        pltpu.VMEM((M,), jnp.float32),
    ],
)
def _sc_warp(x_hbm, nodes_hbm, p_hbm, invd_hbm, out_hbm,
             xbuf, obuf, nbuf, pbuf, ibuf):
    wid = lax.axis_index("s") * 2 + lax.axis_index("c")
    base3 = jnp.minimum(wid * (QPW * 3), (QTOT - QPW) * 3)
    pltpu.sync_copy(x_hbm.at[pl.ds(base3, QPW * 3)], xbuf)
    pltpu.sync_copy(nodes_hbm, nbuf)
    pltpu.sync_copy(p_hbm, pbuf)
    pltpu.sync_copy(invd_hbm, ibuf)

    lanes = lax.broadcasted_iota(jnp.int32, (16,), 0)
    big = jnp.full((16,), jnp.uint32(0xFFFFFFFF), jnp.uint32)
    keep = jnp.uint32(0xFFFFFFFF & ~IDX_MASK)

    def group(g, _):
        q3 = (g * 16 + lanes) * 3
        qx = plsc.load_gather(xbuf, [q3])
        qy = plsc.load_gather(xbuf, [q3 + 1])
        qz = plsc.load_gather(xbuf, [q3 + 2])

        NCH = 4

        def pair_step(qd, carry):
            ms = [list(carry[h * KNN:(h + 1) * KNN]) for h in range(NCH)]
            for h in range(NCH):
                m = ms[h]
                kmin = None
                for j in range(4):
                    n = qd * (4 * NCH) + h * 4 + j
                    dxv = qx - nbuf[pl.ds(n * 48, 16)]
                    dyv = qy - nbuf[pl.ds(n * 48 + 16, 16)]
                    dzv = qz - nbuf[pl.ds(n * 48 + 32, 16)]
                    d2 = dxv * dxv + dyv * dyv + dzv * dzv
                    key = ((plsc.bitcast(d2, jnp.uint32) & keep)
                           | n.astype(jnp.uint32))
                    kmin = key if kmin is None else jnp.minimum(kmin, key)
                for kk in range(KNN):
                    lo = jnp.minimum(m[kk], kmin)
                    kmin = jnp.maximum(m[kk], kmin)
                    m[kk] = lo
            return sum((tuple(ms[h]) for h in range(NCH)), ())

        mm = lax.fori_loop(0, M // (4 * NCH), pair_step,
                           (big,) * (NCH * KNN), unroll=2)
        mA = [jnp.minimum(mm[kk], mm[2 * KNN - 1 - kk]) for kk in range(KNN)]
        mB = [jnp.minimum(mm[2 * KNN + kk], mm[4 * KNN - 1 - kk])
              for kk in range(KNN)]
        m = mA
        for key in mB:
            for kk in range(KNN):
                lo = jnp.minimum(m[kk], key)
                key = jnp.maximum(m[kk], key)
                m[kk] = lo

        cands = []
        for kk in range(KNN):
            base = m[kk] & jnp.uint32(IDX_MASK & ~3)
            for j in range(4):
                gid = (base | jnp.uint32(j)).astype(jnp.int32)
                g48 = gid * 48
                dxv = qx - plsc.load_gather(nbuf, [g48])
                dyv = qy - plsc.load_gather(nbuf, [g48 + 16])
                dzv = qz - plsc.load_gather(nbuf, [g48 + 32])
                d2 = dxv * dxv + dyv * dyv + dzv * dzv
                gk = ((plsc.bitcast(d2, jnp.uint32) & keep)
                      | plsc.bitcast(gid, jnp.uint32))
                cands.append(jnp.where(gk == m[kk], big, gk))
        for key in cands:
            for kk in range(KNN):
                lo = jnp.minimum(m[kk], key)
                key = jnp.maximum(m[kk], key)
                m[kk] = lo

        wsum = jnp.zeros((16,), jnp.float32)
        ax = jnp.zeros((16,), jnp.float32)
        ay = jnp.zeros((16,), jnp.float32)
        az = jnp.zeros((16,), jnp.float32)
        for kk in range(KNN):
            idx = (m[kk] & jnp.uint32(IDX_MASK)).astype(jnp.int32)
            d2a = plsc.bitcast(m[kk] & keep, jnp.float32)
            w = jnp.exp(-d2a * plsc.load_gather(ibuf, [idx]))
            wsum = wsum + w
            i12 = idx * 12
            p0 = plsc.load_gather(pbuf, [i12])
            p1 = plsc.load_gather(pbuf, [i12 + 1])
            p2 = plsc.load_gather(pbuf, [i12 + 2])
            p3 = plsc.load_gather(pbuf, [i12 + 3])
            p4 = plsc.load_gather(pbuf, [i12 + 4])
            p5 = plsc.load_gather(pbuf, [i12 + 5])
            p6 = plsc.load_gather(pbuf, [i12 + 6])
            p7 = plsc.load_gather(pbuf, [i12 + 7])
            p8 = plsc.load_gather(pbuf, [i12 + 8])
            p9 = plsc.load_gather(pbuf, [i12 + 9])
            p10 = plsc.load_gather(pbuf, [i12 + 10])
            p11 = plsc.load_gather(pbuf, [i12 + 11])
            ax = ax + w * (p0 * qx + p1 * qy + p2 * qz + p9)
            ay = ay + w * (p3 * qx + p4 * qy + p5 * qz + p10)
            az = az + w * (p6 * qx + p7 * qy + p8 * qz + p11)
        inv = 1.0 / (wsum + 1e-8)
        plsc.store_scatter(obuf, [q3], ax * inv)
        plsc.store_scatter(obuf, [q3 + 1], ay * inv)
        plsc.store_scatter(obuf, [q3 + 2], az * inv)
        return 0

    lax.fori_loop(0, NGROUP, group, 0)
    pltpu.sync_copy(obuf, out_hbm.at[pl.ds(base3, QPW * 3)])


def kernel(x, t, nodes, node_radius, W0, b0, W1, b1, W2, b2, W3, b3, W4, b4,
           Wout, bout):
    Q = x.shape[0]
    assert Q == QTOT
    P, invd, nrep = _prep(nodes, t, node_radius, W0, b0, W1, b1, W2, b2, W3,
                          b3, W4, b4, Wout, bout)
    warped = _sc_warp(x.reshape(-1), nrep.reshape(-1), P.reshape(-1),
                      invd.reshape(-1))
    return warped.reshape(Q, 3)


# speedup vs baseline: 1.0018x; 1.0018x over previous
import functools

import jax
import jax.numpy as jnp
from jax import lax
from jax.experimental import pallas as pl
from jax.experimental.pallas import tpu as pltpu
from jax.experimental.pallas import tpu_sc as plsc

M = 512
KNN = 8
MRX = 10
MRT = 6

NW = 32
QTOT = 50000
QPW = 1568
NGROUP = QPW // 16

IDX_BITS = 9
IDX_MASK = (1 << IDX_BITS) - 1


def _freq_rows(nfreq):
    i = jax.lax.broadcasted_iota(jnp.int32, (1, nfreq), 1)
    return jnp.exp2(i.astype(jnp.float32))


def _prep_kernel(nodes_ref, t_ref, rrow_ref, W0_ref, b0_ref, W1_ref, b1_ref,
                 W2_ref, b2_ref, W3_ref, b3_ref, W4_ref, b4_ref, Wout_ref,
                 bout_ref, P_ref, invd_ref, nrep_ref):
    nodes = nodes_ref[...]
    nx = nodes[:, 0:1]
    ny = nodes[:, 1:2]
    nz = nodes[:, 2:3]
    fx = _freq_rows(MRX)
    ft = _freq_rows(MRT)
    t = t_ref[0, 0]
    t_col = jnp.full((M, 1), t, jnp.float32)

    parts = [nodes]
    for col in (nx, ny, nz):
        parts.append(jnp.sin(col * fx))
    for col in (nx, ny, nz):
        parts.append(jnp.cos(col * fx))
    parts.append(t_col)
    parts.append(jnp.sin(t_col * ft))
    parts.append(jnp.cos(t_col * ft))
    h0 = jnp.concatenate(parts, axis=1)

    dot = functools.partial(jnp.dot, preferred_element_type=jnp.float32,
                            precision=jax.lax.Precision.HIGHEST)
    h = jax.nn.relu(dot(h0, W0_ref[...]) + b0_ref[...])
    h = jax.nn.relu(dot(h, W1_ref[...]) + b1_ref[...])
    h = jax.nn.relu(dot(h, W2_ref[...]) + b2_ref[...])
    h = jnp.concatenate([h, h0], axis=1)
    h = jax.nn.relu(dot(h, W3_ref[...]) + b3_ref[...])
    h = jax.nn.relu(dot(h, W4_ref[...]) + b4_ref[...])
    out = dot(h, Wout_ref[...]) + bout_ref[...]

    dx = out[:, 0:1]
    dy = out[:, 1:2]
    dz = out[:, 2:3]
    r = out[:, 3:4] + 1.0
    i = out[:, 4:5]
    j = out[:, 5:6]
    k = out[:, 6:7]
    two_s = 2.0 / (r * r + i * i + j * j + k * k + 1e-8)
    R00 = 1.0 - two_s * (j * j + k * k)
    R01 = two_s * (i * j - k * r)
    R02 = two_s * (i * k + j * r)
    R10 = two_s * (i * j + k * r)
    R11 = 1.0 - two_s * (i * i + k * k)
    R12 = two_s * (j * k - i * r)
    R20 = two_s * (i * k - j * r)
    R21 = two_s * (j * k + i * r)
    R22 = 1.0 - two_s * (i * i + j * j)
    cx = nx + dx - (R00 * nx + R01 * ny + R02 * nz)
    cy = ny + dy - (R10 * nx + R11 * ny + R12 * nz)
    cz = nz + dz - (R20 * nx + R21 * ny + R22 * nz)
    P_ref[...] = jnp.concatenate(
        [R00, R01, R02, R10, R11, R12, R20, R21, R22, cx, cy, cz], axis=1)
    rrow = rrow_ref[...]
    invd_ref[...] = 1.0 / (2.0 * rrow * rrow + 1e-8)
    ones = jnp.ones((1, 16), jnp.float32)
    nrep_ref[...] = jnp.concatenate([nx * ones, ny * ones, nz * ones], axis=1)


def _prep(nodes, t, node_radius, W0, b0, W1, b1, W2, b2, W3, b3, W4, b4,
          Wout, bout):
    return pl.pallas_call(
        _prep_kernel,
        out_shape=(jax.ShapeDtypeStruct((M, 12), jnp.float32),
                   jax.ShapeDtypeStruct((1, M), jnp.float32),
                   jax.ShapeDtypeStruct((M, 48), jnp.float32)),
    )(nodes, t.reshape(1, 1), node_radius.reshape(1, -1), W0,
      b0.reshape(1, -1), W1, b1.reshape(1, -1), W2, b2.reshape(1, -1), W3,
      b3.reshape(1, -1), W4, b4.reshape(1, -1), Wout, bout.reshape(1, -1))


@functools.partial(
    pl.kernel,
    mesh=plsc.VectorSubcoreMesh(core_axis_name="c", subcore_axis_name="s"),
    compiler_params=pltpu.CompilerParams(needs_layout_passes=False),
    out_type=jax.ShapeDtypeStruct((QTOT * 3,), jnp.float32),
    scratch_types=[
        pltpu.VMEM((QPW * 3,), jnp.float32),
        pltpu.VMEM((QPW * 3,), jnp.float32),
        pltpu.VMEM((M * 3 * 16,), jnp.float32),
        pltpu.VMEM((M * 12,), jnp.float32),
        pltpu.VMEM((M,), jnp.float32),
    ],
)
def _sc_warp(x_hbm, nodes_hbm, p_hbm, invd_hbm, out_hbm,
             xbuf, obuf, nbuf, pbuf, ibuf):
    wid = lax.axis_index("s") * 2 + lax.axis_index("c")
    base3 = jnp.minimum(wid * (QPW * 3), (QTOT - QPW) * 3)
    pltpu.sync_copy(x_hbm.at[pl.ds(base3, QPW * 3)], xbuf)
    pltpu.sync_copy(nodes_hbm, nbuf)
    pltpu.sync_copy(p_hbm, pbuf)
    pltpu.sync_copy(invd_hbm, ibuf)

    lanes = lax.broadcasted_iota(jnp.int32, (16,), 0)
    big = jnp.full((16,), jnp.uint32(0xFFFFFFFF), jnp.uint32)
    keep = jnp.uint32(0xFFFFFFFF & ~IDX_MASK)

    def group(g, _):
        q3 = (g * 16 + lanes) * 3
        qx = plsc.load_gather(xbuf, [q3])
        qy = plsc.load_gather(xbuf, [q3 + 1])
        qz = plsc.load_gather(xbuf, [q3 + 2])

        def pair_step(qd, carry):
            ms = [list(carry[:KNN]), list(carry[KNN:])]
            for h in range(2):
                m = ms[h]
                kmin = None
                for j in range(4):
                    n = qd * 8 + h * 4 + j
                    dxv = qx - nbuf[pl.ds(n * 48, 16)]
                    dyv = qy - nbuf[pl.ds(n * 48 + 16, 16)]
                    dzv = qz - nbuf[pl.ds(n * 48 + 32, 16)]
                    d2 = dxv * dxv + dyv * dyv + dzv * dzv
                    key = ((plsc.bitcast(d2, jnp.uint32) & keep)
                           | n.astype(jnp.uint32))
                    kmin = key if kmin is None else jnp.minimum(kmin, key)
                for kk in range(KNN):
                    lo = jnp.minimum(m[kk], kmin)
                    kmin = jnp.maximum(m[kk], kmin)
                    m[kk] = lo
            return tuple(ms[0]) + tuple(ms[1])

        mm = lax.fori_loop(0, M // 8, pair_step, (big,) * (2 * KNN), unroll=4)
        m = [jnp.minimum(mm[kk], mm[2 * KNN - 1 - kk]) for kk in range(KNN)]

        cands = []
        for kk in range(KNN):
            base = m[kk] & jnp.uint32(IDX_MASK & ~3)
            for j in range(4):
                gid = (base | jnp.uint32(j)).astype(jnp.int32)
                g48 = gid * 48
                dxv = qx - plsc.load_gather(nbuf, [g48])
                dyv = qy - plsc.load_gather(nbuf, [g48 + 16])
                dzv = qz - plsc.load_gather(nbuf, [g48 + 32])
                d2 = dxv * dxv + dyv * dyv + dzv * dzv
                gk = ((plsc.bitcast(d2, jnp.uint32) & keep)
                      | plsc.bitcast(gid, jnp.uint32))
                cands.append(jnp.where(gk == m[kk], big, gk))
        for key in cands:
            for kk in range(KNN):
                lo = jnp.minimum(m[kk], key)
                key = jnp.maximum(m[kk], key)
                m[kk] = lo

        wsum = jnp.zeros((16,), jnp.float32)
        ax = jnp.zeros((16,), jnp.float32)
        ay = jnp.zeros((16,), jnp.float32)
        az = jnp.zeros((16,), jnp.float32)
        for kk in range(KNN):
            idx = (m[kk] & jnp.uint32(IDX_MASK)).astype(jnp.int32)
            d2a = plsc.bitcast(m[kk] & keep, jnp.float32)
            w = jnp.exp(-d2a * plsc.load_gather(ibuf, [idx]))
            wsum = wsum + w
            i12 = idx * 12
            p0 = plsc.load_gather(pbuf, [i12])
            p1 = plsc.load_gather(pbuf, [i12 + 1])
            p2 = plsc.load_gather(pbuf, [i12 + 2])
            p3 = plsc.load_gather(pbuf, [i12 + 3])
            p4 = plsc.load_gather(pbuf, [i12 + 4])
            p5 = plsc.load_gather(pbuf, [i12 + 5])
            p6 = plsc.load_gather(pbuf, [i12 + 6])
            p7 = plsc.load_gather(pbuf, [i12 + 7])
            p8 = plsc.load_gather(pbuf, [i12 + 8])
            p9 = plsc.load_gather(pbuf, [i12 + 9])
            p10 = plsc.load_gather(pbuf, [i12 + 10])
            p11 = plsc.load_gather(pbuf, [i12 + 11])
            ax = ax + w * (p0 * qx + p1 * qy + p2 * qz + p9)
            ay = ay + w * (p3 * qx + p4 * qy + p5 * qz + p10)
            az = az + w * (p6 * qx + p7 * qy + p8 * qz + p11)
        inv = 1.0 / (wsum + 1e-8)
        plsc.store_scatter(obuf, [q3], ax * inv)
        plsc.store_scatter(obuf, [q3 + 1], ay * inv)
        plsc.store_scatter(obuf, [q3 + 2], az * inv)
        return 0

    lax.fori_loop(0, NGROUP, group, 0)
    pltpu.sync_copy(obuf, out_hbm.at[pl.ds(base3, QPW * 3)])


def kernel(x, t, nodes, node_radius, W0, b0, W1, b1, W2, b2, W3, b3, W4, b4,
           Wout, bout):
    Q = x.shape[0]
    assert Q == QTOT
    P, invd, nrep = _prep(nodes, t, node_radius, W0, b0, W1, b1, W2, b2, W3,
                          b3, W4, b4, Wout, bout)
    warped = _sc_warp(x.reshape(-1), nrep.reshape(-1), P.reshape(-1),
                      invd.reshape(-1))
    return warped.reshape(Q, 3)
